# Initial kernel scaffold; baseline (speedup 1.0000x reference)
import jax
import jax.numpy as jnp
from jax.experimental import pallas as pl


def kernel(speaker_id, emotion_id, spk_table, emo_table, W1, b1, W2, b2):
    raise NotImplementedError("write your pallas kernel here")


# speedup vs baseline: 1.5406x; 1.5406x over previous
import functools

import jax
import jax.numpy as jnp
from jax import lax
from jax.experimental import pallas as pl
from jax.experimental.pallas import tpu as pltpu
from jax.experimental.pallas import tpu_sc as plsc

BATCH = 16384
EMBED = 64
STYLE = 128
N_EMO = 32

NC = 2
NS = 16
NW = NC * NS
B_PER_W = BATCH // NW
IDX_CHUNK = 128
N_CHUNKS = B_PER_W // IDX_CHUNK

BB = 2048
N_BLK = BATCH // BB


def _sc_gather_body(idx_hbm, table_hbm, out_hbm, idx_v, rows_v, sem):
    wid = lax.axis_index("s") * NC + lax.axis_index("c")
    pltpu.sync_copy(idx_hbm.at[wid], idx_v)
    copies = [
        pltpu.async_copy(
            table_hbm.at[idx_v.at[j]],
            rows_v.at[pl.ds(j * IDX_CHUNK, IDX_CHUNK)],
            sem,
        )
        for j in range(N_CHUNKS)
    ]
    for c in copies:
        c.wait()
    pltpu.sync_copy(rows_v, out_hbm.at[pl.ds(wid * B_PER_W, B_PER_W)])


@functools.lru_cache(maxsize=None)
def _make_spk_gather():
    return pl.kernel(
        _sc_gather_body,
        out_type=jax.ShapeDtypeStruct((BATCH, EMBED), jnp.float32),
        mesh=plsc.VectorSubcoreMesh(core_axis_name="c", subcore_axis_name="s",
                                    num_cores=NC, num_subcores=NS),
        scratch_types=[
            pltpu.VMEM((N_CHUNKS, IDX_CHUNK), jnp.int32),
            pltpu.VMEM((B_PER_W, EMBED), jnp.float32),
            pltpu.SemaphoreType.DMA,
        ],
        compiler_params=pltpu.CompilerParams(use_tc_tiling_on_sc=False),
    )


def _mlp_body(spk_ref, eid_ref, emo_ref, w1s_ref, w1e_ref, b1_ref, w2_ref,
              b2_ref, out_ref):
    eid = eid_ref[0, 0, :]
    onehot = (eid[:, None] == lax.broadcasted_iota(jnp.int32, (BB, N_EMO), 1)
              ).astype(jnp.float32)
    emo_proj = jnp.dot(emo_ref[...], w1e_ref[...],
                       preferred_element_type=jnp.float32) + b1_ref[...]
    h = (jnp.dot(spk_ref[...], w1s_ref[...],
                 preferred_element_type=jnp.float32)
         + jnp.dot(onehot, emo_proj, preferred_element_type=jnp.float32))
    h = jnp.maximum(h, 0.0)
    out_ref[...] = jnp.dot(h, w2_ref[...],
                           preferred_element_type=jnp.float32) + b2_ref[...]


def _mlp(spk_emb, eid3, emo_table, w1sT, w1eT, b1, w2T, b2):
    return pl.pallas_call(
        _mlp_body,
        grid=(N_BLK,),
        in_specs=[
            pl.BlockSpec((BB, EMBED), lambda i: (i, 0)),
            pl.BlockSpec((1, 1, BB), lambda i: (i, 0, 0)),
            pl.BlockSpec((N_EMO, EMBED), lambda i: (0, 0)),
            pl.BlockSpec((EMBED, STYLE), lambda i: (0, 0)),
            pl.BlockSpec((EMBED, STYLE), lambda i: (0, 0)),
            pl.BlockSpec((1, STYLE), lambda i: (0, 0)),
            pl.BlockSpec((STYLE, STYLE), lambda i: (0, 0)),
            pl.BlockSpec((1, STYLE), lambda i: (0, 0)),
        ],
        out_specs=pl.BlockSpec((BB, STYLE), lambda i: (i, 0)),
        out_shape=jax.ShapeDtypeStruct((BATCH, STYLE), jnp.float32),
    )(spk_emb, eid3, emo_table, w1sT, w1eT, b1, w2T, b2)


def kernel(speaker_id, emotion_id, spk_table, emo_table, W1, b1, W2, b2):
    idx = speaker_id.astype(jnp.int32).reshape(NW, N_CHUNKS, IDX_CHUNK)
    spk_emb = _make_spk_gather()(idx, spk_table)
    eid3 = emotion_id.astype(jnp.int32).reshape(N_BLK, 1, BB)
    w1sT = W1[:, :EMBED].T
    w1eT = W1[:, EMBED:].T
    out = _mlp(spk_emb, eid3, emo_table, w1sT, w1eT,
               b1.reshape(1, STYLE), W2.T, b2.reshape(1, STYLE))
    return out
